# baseline (speedup 1.0000x reference)
import functools

import jax
import jax.numpy as jnp
from jax import lax
from jax.experimental import pallas as pl
from jax.experimental.pallas import tpu as pltpu
from jax.experimental.pallas import tpu_sc as plsc

C = 64
NX = 512
NY = 512
BATCH = 4
L = 16
NC, NS = 2, 16
NW = NC * NS
KPW = BATCH * NX * NY // NW
XSPAN = KPW // NY
OUT_N = BATCH * C * NX * NY
OUT_PAD = 65536 * 1032
CH = 6144
CAP = 2048
ECH = 32
NDMA = ECH * C // 128
ZROWS = 65536
ZBLK = 8


def _zero_body(o_ref):
    o_ref[...] = jnp.zeros_like(o_ref)


def _sc_body(feat, cb, cy, cx, bsv, canvas,
             cbv, cyv, cxv, bsb, owner, lkbuf, pidbuf,
             rows, oidx, vals, gsem, ssem):
    npad = cb.shape[0]
    nchunks = npad // CH
    wid = lax.axis_index("s") * NC + lax.axis_index("c")
    myb = wid // (NW // BATCH)
    xlo = (wid % (NW // BATCH)) * XSPAN
    gbase = myb * (C * NX * NY) + xlo * NY
    iota = lax.broadcasted_iota(jnp.int32, (L,), 0)
    zi32 = jnp.zeros((L,), jnp.int32)

    def ow_init(i, _):
        owner[pl.ds(i * L, L)] = zi32
        return 0
    lax.fori_loop(0, KPW // L, ow_init, 0)

    def cap_init(i, _):
        lkbuf[pl.ds(i * L, L)] = zi32
        pidbuf[pl.ds(i * L, L)] = zi32
        return 0
    lax.fori_loop(0, CAP // L, cap_init, 0)

    pltpu.sync_copy(bsv, bsb)
    bs_vec = bsb[...]

    for k in range(nchunks):
        pltpu.sync_copy(cb.at[pl.ds(k * CH, CH)], cbv)
        pltpu.sync_copy(cy.at[pl.ds(k * CH, CH)], cyv)
        pltpu.sync_copy(cx.at[pl.ds(k * CH, CH)], cxv)

        def scan(i, _, kbase=k * CH):
            base = i * L
            vb = cbv[pl.ds(base, L)]
            vy = cyv[pl.ds(base, L)]
            vx = cxv[pl.ds(base, L)]
            m = (vb == myb) & (vb < bs_vec) & (vx >= xlo) & (vx < xlo + XSPAN)
            lk = (vx - xlo) * NY + vy
            lk = jnp.where(m, lk, 0)
            pid = kbase + base + iota + 1
            plsc.store_scatter(owner, [lk], pid, mask=m)
            return 0
        lax.fori_loop(0, CH // L, scan, 0)

    def compact(i, cnt):
        v = owner[pl.ds(i * L, L)]
        m = v > 0
        plsc.store_compressed(lkbuf.at[pl.ds(cnt, L)], i * L + iota, mask=m)
        plsc.store_compressed(pidbuf.at[pl.ds(cnt, L)], v - 1, mask=m)
        nc2 = cnt + jnp.sum(m.astype(jnp.int32))
        return jnp.minimum(nc2, CAP - L)
    cnt = lax.fori_loop(0, KPW // L, compact, 0)

    def wchunk(d, _):
        @pl.when(d * ECH < cnt)
        def _():
            pltpu.async_copy(
                feat.at[pidbuf.at[pl.ds(d * ECH, ECH)]], rows, gsem).wait()

            def r2v(i, _):
                rv = plsc.load_gather(
                    rows, [jnp.full((L,), i >> 2, jnp.int32),
                           (i & 3) * L + iota])
                vals[pl.ds(i * L, L)] = rv
                return 0
            lax.fori_loop(0, ECH * C // L, r2v, 0)

            for ev in range(ECH // L):
                off = d * ECH + ev * L
                lkv = lkbuf[pl.ds(off, L)]
                me = (off + iota) < cnt

                def per_ch(c, _):
                    p = ev * (L * C) + iota * C + c
                    spread = wid * (ECH * C) + p
                    ov = jnp.where(me, gbase + c * (NX * NY) + lkv,
                                   OUT_N + spread)
                    plsc.store_scatter(oidx, [p >> 7, p & 127], ov)
                    return 0
                lax.fori_loop(0, C, per_ch, 0)

            def s_issue(dd, _):
                pltpu.async_copy(
                    vals.at[pl.ds(dd * 128, 128)], canvas.at[oidx.at[dd]],
                    ssem)
                return 0
            lax.fori_loop(0, NDMA, s_issue, 0)

            def s_drain(dd, _):
                pltpu.make_async_copy(
                    vals.at[pl.ds(dd * 128, 128)], canvas.at[oidx.at[dd]],
                    ssem).wait()
                return 0
            lax.fori_loop(0, NDMA, s_drain, 0)
        return 0
    lax.fori_loop(0, CAP // ECH, wchunk, 0)


def kernel(voxel_features, coords, batch_size):
    n = coords.shape[0]
    npad = ((n + CH - 1) // CH) * CH
    pad = npad - n
    cb = jnp.concatenate([coords[:, 0], jnp.full((pad,), 511, jnp.int32)])
    cy = jnp.concatenate([coords[:, 2], jnp.zeros((pad,), jnp.int32)])
    cx = jnp.concatenate([coords[:, 3], jnp.zeros((pad,), jnp.int32)])
    bsv = jnp.full((L,), batch_size, dtype=jnp.int32)

    zeros = pl.pallas_call(
        _zero_body,
        out_shape=jax.ShapeDtypeStruct((OUT_PAD // ZROWS, ZROWS),
                                       jnp.float32),
        grid=(OUT_PAD // ZROWS // ZBLK,),
        out_specs=pl.BlockSpec((ZBLK, ZROWS), lambda i: (i, 0)),
    )()

    canvas = jax.new_ref(zeros.reshape(OUT_PAD))

    mesh = plsc.VectorSubcoreMesh(core_axis_name="c", subcore_axis_name="s")
    run = pl.kernel(
        _sc_body,
        out_type=(),
        mesh=mesh,
        compiler_params=pltpu.CompilerParams(needs_layout_passes=False,
                                             use_tc_tiling_on_sc=False),
        scratch_types=[
            pltpu.VMEM((CH,), jnp.int32),
            pltpu.VMEM((CH,), jnp.int32),
            pltpu.VMEM((CH,), jnp.int32),
            pltpu.VMEM((L,), jnp.int32),
            pltpu.VMEM((KPW,), jnp.int32),
            pltpu.VMEM((CAP,), jnp.int32),
            pltpu.VMEM((CAP,), jnp.int32),
            pltpu.VMEM((ECH, C), jnp.float32),
            pltpu.VMEM((NDMA, 128), jnp.int32),
            pltpu.VMEM((ECH * C,), jnp.float32),
            pltpu.SemaphoreType.DMA,
            pltpu.SemaphoreType.DMA,
        ],
    )
    run(voxel_features, cb, cy, cx, bsv, canvas)
    return canvas[...][:OUT_N].reshape(BATCH, C, NX, NY)


# speedup vs baseline: 1.6215x; 1.6215x over previous
import functools

import jax
import jax.numpy as jnp
from jax import lax
from jax.experimental import pallas as pl
from jax.experimental.pallas import tpu as pltpu
from jax.experimental.pallas import tpu_sc as plsc

C = 64
NX = 512
NY = 512
BATCH = 4
L = 16
NC, NS = 2, 16
NW = NC * NS
KPW = BATCH * NX * NY // NW
XSPAN = KPW // NY
OUT_N = BATCH * C * NX * NY
CH = 6144
CAP = 2048
ECH = 32
NDMA = ECH * C // 128
ZBLK = 1 << 21


def _zero_body(o_ref):
    o_ref[...] = jnp.zeros_like(o_ref)


def _sc_body(feat, cb, cy, cx, bsv, canvas,
             cbv, cyv, cxv, bsb, owner, lkbuf, pidbuf,
             rows, oidx, vals, gsem, ssem):
    npad = cb.shape[0]
    nchunks = npad // CH
    wid = lax.axis_index("s") * NC + lax.axis_index("c")
    myb = wid // (NW // BATCH)
    xlo = (wid % (NW // BATCH)) * XSPAN
    gbase = myb * (C * NX * NY) + xlo * NY
    iota = lax.broadcasted_iota(jnp.int32, (L,), 0)
    zi32 = jnp.zeros((L,), jnp.int32)

    def ow_init(i, _):
        owner[pl.ds(i * L, L)] = zi32
        return 0
    lax.fori_loop(0, KPW // L, ow_init, 0)

    def cap_init(i, _):
        lkbuf[pl.ds(i * L, L)] = zi32
        pidbuf[pl.ds(i * L, L)] = zi32
        return 0
    lax.fori_loop(0, CAP // L + 2, cap_init, 0)

    pltpu.sync_copy(bsv, bsb)
    bs_vec = bsb[...]

    for k in range(nchunks):
        pltpu.sync_copy(cb.at[pl.ds(k * CH, CH)], cbv)
        pltpu.sync_copy(cy.at[pl.ds(k * CH, CH)], cyv)
        pltpu.sync_copy(cx.at[pl.ds(k * CH, CH)], cxv)

        def scan(i, _, kbase=k * CH):
            base = i * L
            vb = cbv[pl.ds(base, L)]
            vy = cyv[pl.ds(base, L)]
            vx = cxv[pl.ds(base, L)]
            m = (vb == myb) & (vb < bs_vec) & (vx >= xlo) & (vx < xlo + XSPAN)
            lk = (vx - xlo) * NY + vy
            lk = jnp.where(m, lk, 0)
            pid = kbase + base + iota + 1
            plsc.store_scatter(owner, [lk], pid, mask=m)
            return 0
        lax.fori_loop(0, CH // L, scan, 0)

    def compact(i, cnt):
        v = owner[pl.ds(i * L, L)]
        m = v > 0
        plsc.store_compressed(lkbuf.at[pl.ds(cnt, L)], i * L + iota, mask=m)
        plsc.store_compressed(pidbuf.at[pl.ds(cnt, L)], v - 1, mask=m)
        nc2 = cnt + jnp.sum(m.astype(jnp.int32))
        return jnp.minimum(nc2, CAP - L)
    cnt = lax.fori_loop(0, KPW // L, compact, 0)

    @pl.when(cnt > 0)
    def _():
        lk0 = plsc.load_gather(lkbuf, [zi32])
        pid0 = plsc.load_gather(pidbuf, [zi32])
        lkbuf[pl.ds(cnt, L)] = lk0
        pidbuf[pl.ds(cnt, L)] = pid0
        lkbuf[pl.ds(cnt + L, L)] = lk0
        pidbuf[pl.ds(cnt + L, L)] = pid0

    def wchunk(d, _):
        @pl.when(d * ECH < cnt)
        def _():
            pltpu.async_copy(
                feat.at[pidbuf.at[pl.ds(d * ECH, ECH)]], rows, gsem).wait()

            def r2v(i, _):
                rv = plsc.load_gather(
                    rows, [jnp.full((L,), i >> 2, jnp.int32),
                           (i & 3) * L + iota])
                vals[pl.ds(i * L, L)] = rv
                return 0
            lax.fori_loop(0, ECH * C // L, r2v, 0)

            for ev in range(ECH // L):
                off = d * ECH + ev * L
                lkv = lkbuf[pl.ds(off, L)]

                def per_ch(c, _):
                    p = ev * (L * C) + iota * C + c
                    ov = gbase + c * (NX * NY) + lkv
                    plsc.store_scatter(oidx, [p >> 7, p & 127], ov)
                    return 0
                lax.fori_loop(0, C, per_ch, 0)

            def s_issue(dd, _):
                pltpu.async_copy(
                    vals.at[pl.ds(dd * 128, 128)], canvas.at[oidx.at[dd]],
                    ssem)
                return 0
            lax.fori_loop(0, NDMA, s_issue, 0)

            def s_drain(dd, _):
                pltpu.make_async_copy(
                    vals.at[pl.ds(dd * 128, 128)], canvas.at[oidx.at[dd]],
                    ssem).wait()
                return 0
            lax.fori_loop(0, NDMA, s_drain, 0)
        return 0
    lax.fori_loop(0, CAP // ECH, wchunk, 0)


def kernel(voxel_features, coords, batch_size):
    n = coords.shape[0]
    npad = ((n + CH - 1) // CH) * CH
    pad = npad - n
    cb = jnp.concatenate([coords[:, 0], jnp.full((pad,), 511, jnp.int32)])
    cy = jnp.concatenate([coords[:, 2], jnp.zeros((pad,), jnp.int32)])
    cx = jnp.concatenate([coords[:, 3], jnp.zeros((pad,), jnp.int32)])
    bsv = jnp.full((L,), batch_size, dtype=jnp.int32)

    zeros = pl.pallas_call(
        _zero_body,
        out_shape=jax.ShapeDtypeStruct((OUT_N,), jnp.float32),
        grid=(OUT_N // ZBLK,),
        out_specs=pl.BlockSpec((ZBLK,), lambda i: (i,)),
    )()

    canvas = jax.new_ref(zeros)

    mesh = plsc.VectorSubcoreMesh(core_axis_name="c", subcore_axis_name="s")
    run = pl.kernel(
        _sc_body,
        out_type=(),
        mesh=mesh,
        compiler_params=pltpu.CompilerParams(needs_layout_passes=False,
                                             use_tc_tiling_on_sc=False),
        scratch_types=[
            pltpu.VMEM((CH,), jnp.int32),
            pltpu.VMEM((CH,), jnp.int32),
            pltpu.VMEM((CH,), jnp.int32),
            pltpu.VMEM((L,), jnp.int32),
            pltpu.VMEM((KPW,), jnp.int32),
            pltpu.VMEM((CAP + 2 * L,), jnp.int32),
            pltpu.VMEM((CAP + 2 * L,), jnp.int32),
            pltpu.VMEM((ECH, C), jnp.float32),
            pltpu.VMEM((NDMA, 128), jnp.int32),
            pltpu.VMEM((ECH * C,), jnp.float32),
            pltpu.SemaphoreType.DMA,
            pltpu.SemaphoreType.DMA,
        ],
    )
    run(voxel_features, cb, cy, cx, bsv, canvas)
    return canvas[...].reshape(BATCH, C, NX, NY)
